# Initial kernel scaffold; baseline (speedup 1.0000x reference)
import functools

import jax
import jax.numpy as jnp
from jax import lax
from jax.experimental import pallas as pl
from jax.experimental.pallas import tpu as pltpu
from jax.experimental.pallas import tpu_sc as plsc


@functools.lru_cache(maxsize=None)
def _build_gather(B: int, D: int):
    info = plsc.get_sparse_core_info()
    NC, NS = info.num_cores, info.num_subcores
    NW = NC * NS
    assert B % NW == 0
    b_per_w = B // NW
    CHUNK = 3200
    assert b_per_w % CHUNK == 0
    n_chunks = b_per_w // CHUNK

    mesh = plsc.VectorSubcoreMesh(core_axis_name="c", subcore_axis_name="s")

    @functools.partial(
        pl.kernel,
        mesh=mesh,
        out_type=jax.ShapeDtypeStruct((B, D), jnp.float32),
        scratch_types=[
            pltpu.VMEM((CHUNK,), jnp.int32),
            pltpu.VMEM((CHUNK, D), jnp.float32),
            pltpu.SemaphoreType.DMA,
        ],
    )
    def gather_kernel(idx_hbm, table_hbm, out_hbm, idx_v, rows_v, sem):
        wid = lax.axis_index("s") * NC + lax.axis_index("c")
        base = wid * b_per_w

        @pl.loop(0, n_chunks)
        def _(i):
            off = base + i * CHUNK
            pltpu.sync_copy(idx_hbm.at[pl.ds(off, CHUNK)], idx_v)
            pltpu.async_copy(table_hbm.at[idx_v], rows_v, sem).wait()
            pltpu.sync_copy(rows_v, out_hbm.at[pl.ds(off, CHUNK)])

    return gather_kernel


def kernel(x, weight):
    B0, S = x.shape
    D = weight.shape[1]
    flat = x.reshape(B0 * S).astype(jnp.int32)
    out = _build_gather(B0 * S, D)(flat, weight)
    return out.reshape(B0, S, D)


# speedup vs baseline: 1.1113x; 1.1113x over previous
import functools

import jax
import jax.numpy as jnp
from jax import lax
from jax.experimental import pallas as pl
from jax.experimental.pallas import tpu as pltpu
from jax.experimental.pallas import tpu_sc as plsc


@functools.lru_cache(maxsize=None)
def _build_gather(B: int, D: int):
    info = plsc.get_sparse_core_info()
    NC, NS = info.num_cores, info.num_subcores
    NW = NC * NS
    assert B % NW == 0
    b_per_w = B // NW
    CHUNK = 3200
    assert b_per_w % CHUNK == 0
    n_chunks = b_per_w // CHUNK

    mesh = plsc.VectorSubcoreMesh(core_axis_name="c", subcore_axis_name="s")

    @functools.partial(
        pl.kernel,
        mesh=mesh,
        out_type=jax.ShapeDtypeStruct((B, D), jnp.float32),
        scratch_types=[
            pltpu.VMEM((CHUNK,), jnp.int32),
            pltpu.VMEM((CHUNK, D), jnp.float32),
            pltpu.SemaphoreType.DMA,
        ],
        compiler_params=pltpu.CompilerParams(use_tc_tiling_on_sc=False),
    )
    def gather_kernel(idx_hbm, table_hbm, out_hbm, idx_v, rows_v, sem):
        wid = lax.axis_index("s") * NC + lax.axis_index("c")
        base = wid * b_per_w

        @pl.loop(0, n_chunks)
        def _(i):
            off = base + i * CHUNK
            pltpu.sync_copy(idx_hbm.at[pl.ds(off, CHUNK)], idx_v)
            pltpu.async_copy(table_hbm.at[idx_v], rows_v, sem).wait()
            pltpu.sync_copy(rows_v, out_hbm.at[pl.ds(off, CHUNK)])

    return gather_kernel


def kernel(x, weight):
    B0, S = x.shape
    D = weight.shape[1]
    flat = x.reshape(B0 * S).astype(jnp.int32)
    out = _build_gather(B0 * S, D)(flat, weight)
    return out.reshape(B0, S, D)


# speedup vs baseline: 1.5251x; 1.3724x over previous
import functools

import jax
import jax.numpy as jnp
from jax import lax
from jax.experimental import pallas as pl
from jax.experimental.pallas import tpu as pltpu
from jax.experimental.pallas import tpu_sc as plsc

_S = 50
_SB = 10
_NSB = _S // _SB
_L = 128
_Q = 4
_I = 8
_D = 32
_CH = _L * _SB


@functools.lru_cache(maxsize=None)
def _build(NB: int, V: int):
    B = NB * _S
    info = plsc.get_sparse_core_info()
    NC, NS = info.num_cores, info.num_subcores
    NW = NC * NS
    J = NB // _L
    JW = J // NW
    b_per_w = B // NW

    mesh = plsc.VectorSubcoreMesh(core_axis_name="c", subcore_axis_name="s")

    @functools.partial(
        pl.kernel,
        mesh=mesh,
        out_type=jax.ShapeDtypeStruct((_S, _Q, J, _I, _L), jnp.float32),
        scratch_types=[
            pltpu.VMEM((b_per_w,), jnp.int32),
            pltpu.VMEM((_CH,), jnp.int32),
            pltpu.VMEM((_CH, _D), jnp.float32),
            pltpu.VMEM((_SB, _Q, 1, _I, _L), jnp.float32),
            pltpu.SemaphoreType.DMA,
        ],
        compiler_params=pltpu.CompilerParams(
            use_tc_tiling_on_sc=False, needs_layout_passes=False),
    )
    def gather_kernel(idx_hbm, table_hbm, out_hbm, idx_all, idx_c, rows_v,
                      out_t, sem):
        wid = lax.axis_index("s") * NC + lax.axis_index("c")
        base = wid * b_per_w
        pltpu.sync_copy(idx_hbm.at[pl.ds(base, b_per_w)], idx_all)

        iota = lax.iota(jnp.int32, 16)
        iota_sb = iota * _SB
        iota_s = iota * _S

        @pl.loop(0, JW)
        def _(jj):
            @pl.loop(0, _NSB)
            def _(sb):
                s0 = sb * _SB
                j = wid * JW + jj

                @pl.loop(0, _SB)
                def _(ss):
                    for bg in range(_L // 16):
                        vpos = iota_s + ((jj * _L + bg * 16) * _S + s0 + ss)
                        vals = plsc.load_gather(idx_all, [vpos])
                        vdst = iota_sb + (bg * 16 * _SB + ss)
                        plsc.store_scatter(idx_c, [vdst], vals)

                pltpu.async_copy(table_hbm.at[idx_c], rows_v, sem).wait()

                @pl.loop(0, _SB)
                def _(ss):
                    @pl.loop(0, _D)
                    def _(c):
                        q = c // _I
                        i = c - q * _I
                        vcol = lax.broadcast(c, (16,))
                        for g in range(_L // 16):
                            vrow = iota_sb + (g * 16 * _SB + ss)
                            out_t[ss, q, 0, i, pl.ds(g * 16, 16)] = (
                                plsc.load_gather(rows_v, [vrow, vcol]))

                pltpu.sync_copy(
                    out_t, out_hbm.at[pl.ds(s0, _SB), :, pl.ds(j, 1), :, :])

    return gather_kernel


def kernel(x, weight):
    NB, S = x.shape
    V, D = weight.shape
    flat = x.reshape(NB * S).astype(jnp.int32)
    out5 = _build(NB, V)(flat, weight)
    return jnp.transpose(out5, (2, 4, 0, 1, 3)).reshape(NB, S, D)


# speedup vs baseline: 1.5338x; 1.0057x over previous
import functools

import jax
import jax.numpy as jnp
from jax import lax
from jax.experimental import pallas as pl
from jax.experimental.pallas import tpu as pltpu
from jax.experimental.pallas import tpu_sc as plsc

_S = 50
_SB = 10
_NSB = _S // _SB
_L = 128
_Q = 4
_I = 8
_D = 32
_CH = _L * _SB


@functools.lru_cache(maxsize=None)
def _build(NB: int, V: int):
    B = NB * _S
    info = plsc.get_sparse_core_info()
    NC, NS = info.num_cores, info.num_subcores
    NW = NC * NS
    J = NB // _L
    JW = J // NW
    b_per_w = B // NW

    mesh = plsc.VectorSubcoreMesh(core_axis_name="c", subcore_axis_name="s")

    @functools.partial(
        pl.kernel,
        mesh=mesh,
        out_type=jax.ShapeDtypeStruct((_S, _Q, J, _I, _L), jnp.float32),
        scratch_types=[
            pltpu.VMEM((b_per_w,), jnp.int32),
            pltpu.VMEM((_CH,), jnp.int32),
            pltpu.VMEM((_CH, _D), jnp.float32),
            pltpu.VMEM((_SB, _Q, 1, _I, _L), jnp.float32),
            pltpu.SemaphoreType.DMA,
        ],
        compiler_params=pltpu.CompilerParams(
            use_tc_tiling_on_sc=False, needs_layout_passes=False),
    )
    def gather_kernel(idx_hbm, table_hbm, out_hbm, idx_all, idx_c, rows_v,
                      out_t, sem):
        wid = lax.axis_index("s") * NC + lax.axis_index("c")
        base = wid * b_per_w
        pltpu.sync_copy(idx_hbm.at[pl.ds(base, b_per_w)], idx_all)

        iota = lax.iota(jnp.int32, 16)
        iota_sb = iota * _SB
        iota_s = iota * _S

        @pl.loop(0, JW)
        def _(jj):
            j = wid * JW + jj

            @pl.loop(0, _NSB)
            def _(sb):
                s0 = sb * _SB

                cbase = jj * _L * _S + s0
                for ss in range(_SB):
                    for bg in range(_L // 16):
                        vpos = iota_s + (cbase + bg * 16 * _S + ss)
                        vals = plsc.load_gather(idx_all, [vpos])
                        vdst = iota_sb + (bg * 16 * _SB + ss)
                        plsc.store_scatter(idx_c, [vdst], vals)

                pltpu.async_copy(table_hbm.at[idx_c], rows_v, sem).wait()

                @pl.loop(0, _SB)
                def _(ss):
                    vss = lax.broadcast(ss, (16,))
                    for c in range(_D):
                        q, i = c // _I, c % _I
                        vcc = lax.broadcast(jnp.int32(c), (16,))
                        for g in range(_L // 16):
                            vrow = iota_sb + (g * 16 * _SB)
                            out_t[ss, q, 0, i, pl.ds(g * 16, 16)] = (
                                plsc.load_gather(
                                    rows_v, [vrow + vss, vcc]))

                pltpu.sync_copy(
                    out_t, out_hbm.at[pl.ds(s0, _SB), :, pl.ds(j, 1), :, :])

    return gather_kernel


def kernel(x, weight):
    NB, S = x.shape
    V, D = weight.shape
    flat = x.reshape(NB * S).astype(jnp.int32)
    out5 = _build(NB, V)(flat, weight)
    return jnp.transpose(out5, (2, 4, 0, 1, 3)).reshape(NB, S, D)


# speedup vs baseline: 2.3976x; 1.5632x over previous
import functools

import jax
import jax.numpy as jnp
from jax import lax
from jax.experimental import pallas as pl
from jax.experimental.pallas import tpu as pltpu
from jax.experimental.pallas import tpu_sc as plsc

_S = 50
_SB = 10
_NSB = _S // _SB
_L = 128
_Q = 4
_I = 8
_D = 32
_CH = _L * _SB


@functools.lru_cache(maxsize=None)
def _build(NB: int, V: int):
    B = NB * _S
    info = plsc.get_sparse_core_info()
    NC, NS = info.num_cores, info.num_subcores
    NW = NC * NS
    J = NB // _L
    JW = J // NW
    b_per_w = B // NW

    mesh = plsc.VectorSubcoreMesh(core_axis_name="c", subcore_axis_name="s")

    @functools.partial(
        pl.kernel,
        mesh=mesh,
        out_type=jax.ShapeDtypeStruct((_S, _Q, J, _I, _L), jnp.float32),
        scratch_types=[
            pltpu.VMEM((b_per_w,), jnp.int32),
            pltpu.VMEM((_CH,), jnp.int32),
            pltpu.VMEM((_CH, _D), jnp.float32),
            pltpu.VMEM((_SB, _Q, 1, _I, _L + 1), jnp.float32),
            pltpu.SemaphoreType.DMA,
        ],
        compiler_params=pltpu.CompilerParams(
            use_tc_tiling_on_sc=False, needs_layout_passes=False),
    )
    def gather_kernel(idx_hbm, table_hbm, out_hbm, idx_all, idx_c, rows_v,
                      out_t, sem):
        wid = lax.axis_index("s") * NC + lax.axis_index("c")
        base = wid * b_per_w
        pltpu.sync_copy(idx_hbm.at[pl.ds(base, b_per_w)], idx_all)

        iota = lax.iota(jnp.int32, 16)
        iota_sb = iota * _SB
        iota_s = iota * _S
        vzero = lax.broadcast(jnp.int32(0), (16,))

        @pl.loop(0, JW)
        def _(jj):
            j = wid * JW + jj

            @pl.loop(0, _NSB)
            def _(sb):
                s0 = sb * _SB

                cbase = jj * _L * _S + s0
                for ss in range(_SB):
                    for bg in range(_L // 16):
                        vpos = iota_s + (cbase + bg * 16 * _S + ss)
                        vals = plsc.load_gather(idx_all, [vpos])
                        vdst = iota_sb + (bg * 16 * _SB + ss)
                        plsc.store_scatter(idx_c, [vdst], vals)

                pltpu.async_copy(table_hbm.at[idx_c], rows_v, sem).wait()

                @pl.loop(0, _L)
                def _(bb):
                    vbb = lax.broadcast(bb, (16,))
                    for ss in range(_SB):
                        k = bb * _SB + ss
                        vss = lax.broadcast(jnp.int32(ss), (16,))
                        for cg in range(_D // 16):
                            vals = rows_v[k, pl.ds(cg * 16, 16)]
                            vc = iota + cg * 16
                            vq = lax.shift_right_logical(vc, 3)
                            vi = lax.bitwise_and(vc, 7)
                            plsc.store_scatter(
                                out_t, [vss, vq, vzero, vi, vbb], vals)

                pltpu.sync_copy(
                    out_t.at[:, :, :, :, pl.ds(0, _L)],
                    out_hbm.at[pl.ds(s0, _SB), :, pl.ds(j, 1), :, :])

    return gather_kernel


def kernel(x, weight):
    NB, S = x.shape
    V, D = weight.shape
    flat = x.reshape(NB * S).astype(jnp.int32)
    out5 = _build(NB, V)(flat, weight)
    return jnp.transpose(out5, (2, 4, 0, 1, 3)).reshape(NB, S, D)
